# baseline (speedup 1.0000x reference)
import functools

import jax
import jax.numpy as jnp
from jax import lax
from jax.experimental import pallas as pl
from jax.experimental.pallas import tpu as pltpu
from jax.experimental.pallas import tpu_sc as plsc

N = 10000
E = 320000
D_IN = 128
H = 8
F = 16
HF = H * F
TW = HF + 16

NC = 2
NS = 16
NW = NC * NS
K = 80
SCW = 10
NCH = 130
NSC = NCH // SCW
EPW_P = NCH * K
SRCROWS = NW * NCH
NPAD = N + K
NZCH = NPAD // K
CHK_R = 400
NCHK = N // CHK_R

BLK = 1000
GRID = N // BLK


def _pro_body(x_ref, wp_ref, ws_ref, asrc_ref, atrg_ref,
              tp_ref, skip_ref, strg_ref, maxs_ref, maxt_ref):
    xb = x_ref[...]
    p = jnp.dot(xb, wp_ref[...], preferred_element_type=jnp.float32)
    skip_ref[...] = jnp.dot(xb, ws_ref[...], preferred_element_type=jnp.float32)
    ss = jnp.dot(p, asrc_ref[...], preferred_element_type=jnp.float32)
    st = jnp.dot(p, atrg_ref[...], preferred_element_type=jnp.float32)
    tp_ref[...] = jnp.concatenate([p, ss], axis=1)
    strg_ref[...] = st
    maxs_ref[...] = jnp.max(ss, axis=0).reshape(1, 1, 16)
    maxt_ref[...] = jnp.max(st, axis=0).reshape(1, 1, 16)


_prologue = pl.pallas_call(
    _pro_body,
    grid=(GRID,),
    in_specs=[
        pl.BlockSpec((BLK, D_IN), lambda i: (i, 0)),
        pl.BlockSpec((D_IN, HF), lambda i: (0, 0)),
        pl.BlockSpec((D_IN, HF), lambda i: (0, 0)),
        pl.BlockSpec((HF, 16), lambda i: (0, 0)),
        pl.BlockSpec((HF, 16), lambda i: (0, 0)),
    ],
    out_specs=[
        pl.BlockSpec((BLK, TW), lambda i: (i, 0)),
        pl.BlockSpec((BLK, HF), lambda i: (i, 0)),
        pl.BlockSpec((BLK, 16), lambda i: (i, 0)),
        pl.BlockSpec((1, 1, 16), lambda i: (i, 0, 0)),
        pl.BlockSpec((1, 1, 16), lambda i: (i, 0, 0)),
    ],
    out_shape=[
        jax.ShapeDtypeStruct((N, TW), jnp.float32),
        jax.ShapeDtypeStruct((N, HF), jnp.float32),
        jax.ShapeDtypeStruct((N, 16), jnp.float32),
        jax.ShapeDtypeStruct((GRID, 1, 16), jnp.float32),
        jax.ShapeDtypeStruct((GRID, 1, 16), jnp.float32),
    ],
)


def _sc_body(ei_hbm, tp_hbm, strg_hbm, m_hbm,
             accp_hbm,
             acc_sh, sblk, tblk, tpv0, tpv1, stv0, stv1, wov, mv,
             semA, semB):
    c = lax.axis_index("c")
    s = lax.axis_index("s")
    w = s * NC + c
    tpvs, stvs, sems = (tpv0, tpv1), (stv0, stv1), (semA, semB)

    z16 = jnp.zeros((16,), jnp.float32)

    def zrow(r, _):
        for j in range(TW // 16):
            wov[r, pl.ds(j * 16, 16)] = z16
        return _

    lax.fori_loop(0, K, zrow, None)
    for b in range(-(-NZCH // NS)):
        cid = s + NS * b
        @pl.when(cid < NZCH)
        def _():
            ro = pl.multiple_of(cid * K, 8)
            pltpu.sync_copy(wov, acc_sh.at[pl.ds(ro, K)])
    pltpu.sync_copy(m_hbm, mv)
    plsc.subcore_barrier()

    iota16 = lax.iota(jnp.int32, 16)
    zero_i = iota16 * 0
    hidx = [(zero_i + h)[:, None] for h in range(H)]
    gdn = lax.GatherDimensionNumbers(
        offset_dims=(), collapsed_slice_dims=(0,), start_index_map=(0,))

    srow0 = w * NCH
    trow0 = SRCROWS + w * NCH

    def issue(lc, b):
        pltpu.async_copy(tp_hbm.at[sblk.at[lc]], tpvs[b], sems[b])
        pltpu.async_copy(strg_hbm.at[tblk.at[lc]], stvs[b], sems[b])

    def drain(lc, b):
        pltpu.make_async_copy(tp_hbm.at[sblk.at[lc]], tpvs[b], sems[b]).wait()
        pltpu.make_async_copy(strg_hbm.at[tblk.at[lc]], stvs[b], sems[b]).wait()

    def copy_idx(t):
        pltpu.sync_copy(ei_hbm.at[pl.ds(srow0 + t * SCW, SCW)], sblk)
        pltpu.sync_copy(ei_hbm.at[pl.ds(trow0 + t * SCW, SCW)], tblk)

    def compute(b):
        tpv, stv = tpvs[b], stvs[b]
        mreg = mv[...]

        def edge4(j4, _):
            for u in range(4):
                j = j4 * 4 + u
                z = tpv[j, pl.ds(HF, 16)] + stv[j, :]
                z = jnp.maximum(z, z * 0.2) - mreg
                e = jnp.exp(z)
                wov[j, pl.ds(HF, 16)] = e
                for h in range(H):
                    sp = lax.gather(
                        e, hidx[h], gdn, slice_sizes=(1,),
                        mode=lax.GatherScatterMode.PROMISE_IN_BOUNDS)
                    wov[j, pl.ds(h * 16, 16)] = tpv[j, pl.ds(h * 16, 16)] * sp
            return _

        lax.fori_loop(0, K // 4, edge4, None)

    copy_idx(0)
    issue(0, 0)

    def superchunk(t, _):
        for lc in range(SCW):
            b = lc % 2
            drain(lc, b)
            if lc + 1 < SCW:
                issue(lc + 1, 1 - b)
            compute(b)
            pltpu.sync_copy(wov, acc_sh.at[tblk.at[lc]], add=True)
            if lc + 1 == SCW:
                @pl.when(t + 1 < NSC)
                def _():
                    copy_idx(t + 1)
                    issue(0, 1 - b)
        return _

    lax.fori_loop(0, NSC, superchunk, None)
    plsc.subcore_barrier()

    for b in range(2):
        cid = s + NS * b
        if (NS * b) < NCHK:
            @pl.when(cid < NCHK)
            def _():
                ro = pl.multiple_of(cid * CHK_R, 8)
                pltpu.sync_copy(acc_sh.at[pl.ds(ro, CHK_R)],
                                accp_hbm.at[c, pl.ds(ro, CHK_R)])


_sc_edge = functools.partial(
    pl.kernel,
    out_type=jax.ShapeDtypeStruct((NC, N, TW), jnp.float32),
    mesh=plsc.VectorSubcoreMesh(core_axis_name="c", subcore_axis_name="s"),
    compiler_params=pltpu.CompilerParams(use_tc_tiling_on_sc=False),
    scratch_types=[
        pltpu.VMEM_SHARED((NPAD, TW), jnp.float32),
        pltpu.VMEM((SCW, K), jnp.int32),
        pltpu.VMEM((SCW, K), jnp.int32),
        pltpu.VMEM((K, TW), jnp.float32),
        pltpu.VMEM((K, TW), jnp.float32),
        pltpu.VMEM((K, 16), jnp.float32),
        pltpu.VMEM((K, 16), jnp.float32),
        pltpu.VMEM((K, TW), jnp.float32),
        pltpu.VMEM((16,), jnp.float32),
        pltpu.SemaphoreType.DMA,
        pltpu.SemaphoreType.DMA,
    ],
)(_sc_body)


def _epi_body(accp_ref, skip_ref, bias_ref, sel_ref, out_ref):
    a = accp_ref[0] + accp_ref[1]
    o = a[:, :HF]
    d = a[:, HF:]
    dexp = jnp.dot(d, sel_ref[...], preferred_element_type=jnp.float32) + 1e-16
    z = o / dexp + skip_ref[...] + bias_ref[...]
    out_ref[...] = jnp.where(z > 0, z, jnp.exp(jnp.minimum(z, 0.0)) - 1.0)


_epilogue = pl.pallas_call(
    _epi_body,
    grid=(GRID,),
    in_specs=[
        pl.BlockSpec((NC, BLK, TW), lambda i: (0, i, 0)),
        pl.BlockSpec((BLK, HF), lambda i: (i, 0)),
        pl.BlockSpec((1, HF), lambda i: (0, 0)),
        pl.BlockSpec((16, HF), lambda i: (0, 0)),
    ],
    out_specs=pl.BlockSpec((BLK, HF), lambda i: (i, 0)),
    out_shape=jax.ShapeDtypeStruct((N, HF), jnp.float32),
)


def kernel(x, edge_index, W_proj, a_src, a_trg, W_skip, bias):
    f32 = jnp.float32
    rows = jnp.arange(HF)
    cols = rows // F
    asrc_m = jnp.zeros((HF, 16), f32).at[rows, cols].set(a_src.reshape(HF))
    atrg_m = jnp.zeros((HF, 16), f32).at[rows, cols].set(a_trg.reshape(HF))
    sel16 = jnp.zeros((16, HF), f32).at[cols, rows].set(1.0)

    tp, skip, strg16, maxs, maxt = _prologue(
        x, W_proj.T, W_skip.T, asrc_m, atrg_m)

    msum = jnp.max(maxs) + jnp.max(maxt)
    m = jnp.maximum(msum, 0.2 * msum)
    m16 = jnp.full((16,), m, f32)

    npad = EPW_P - E // NW
    srcp = jnp.concatenate(
        [edge_index[0].reshape(NW, E // NW),
         jnp.zeros((NW, npad), jnp.int32)], axis=1)
    trgp = jnp.concatenate(
        [edge_index[1].reshape(NW, E // NW),
         jnp.full((NW, npad), N, jnp.int32)], axis=1)
    ei2 = jnp.concatenate([srcp.reshape(-1, K), trgp.reshape(-1, K)], axis=0)

    accp = _sc_edge(ei2, tp, strg16, m16)

    out = _epilogue(accp, skip, bias.reshape(1, HF), sel16)
    return (out, edge_index)


# speedup vs baseline: 1.0005x; 1.0005x over previous
import functools

import jax
import jax.numpy as jnp
from jax import lax
from jax.experimental import pallas as pl
from jax.experimental.pallas import tpu as pltpu
from jax.experimental.pallas import tpu_sc as plsc

N = 10000
E = 320000
D_IN = 128
H = 8
F = 16
HF = H * F
TW = HF + 16

NC = 2
NS = 16
NW = NC * NS
K = 80
SCW = 10
NCH = 130
NSC = NCH // SCW
EPW_P = NCH * K
SRCROWS = NW * NCH
NPAD = N + K
NZCH = NPAD // K
CHK_R = 400
NCHK = N // CHK_R

BLK = 1000
GRID = N // BLK


def _pro_body(x_ref, wp_ref, ws_ref, asrc_ref, atrg_ref,
              tp_ref, skip_ref, strg_ref, maxs_ref, maxt_ref):
    xb = x_ref[...]
    p = jnp.dot(xb, wp_ref[...], preferred_element_type=jnp.float32)
    skip_ref[...] = jnp.dot(xb, ws_ref[...], preferred_element_type=jnp.float32)
    ss = jnp.dot(p, asrc_ref[...], preferred_element_type=jnp.float32)
    st = jnp.dot(p, atrg_ref[...], preferred_element_type=jnp.float32)
    tp_ref[...] = jnp.concatenate([p, ss], axis=1)
    strg_ref[...] = st
    maxs_ref[...] = jnp.max(ss, axis=0).reshape(1, 1, 16)
    maxt_ref[...] = jnp.max(st, axis=0).reshape(1, 1, 16)


_prologue = pl.pallas_call(
    _pro_body,
    grid=(GRID,),
    in_specs=[
        pl.BlockSpec((BLK, D_IN), lambda i: (i, 0)),
        pl.BlockSpec((D_IN, HF), lambda i: (0, 0)),
        pl.BlockSpec((D_IN, HF), lambda i: (0, 0)),
        pl.BlockSpec((HF, 16), lambda i: (0, 0)),
        pl.BlockSpec((HF, 16), lambda i: (0, 0)),
    ],
    out_specs=[
        pl.BlockSpec((BLK, TW), lambda i: (i, 0)),
        pl.BlockSpec((BLK, HF), lambda i: (i, 0)),
        pl.BlockSpec((BLK, 16), lambda i: (i, 0)),
        pl.BlockSpec((1, 1, 16), lambda i: (i, 0, 0)),
        pl.BlockSpec((1, 1, 16), lambda i: (i, 0, 0)),
    ],
    out_shape=[
        jax.ShapeDtypeStruct((N, TW), jnp.float32),
        jax.ShapeDtypeStruct((N, HF), jnp.float32),
        jax.ShapeDtypeStruct((N, 16), jnp.float32),
        jax.ShapeDtypeStruct((GRID, 1, 16), jnp.float32),
        jax.ShapeDtypeStruct((GRID, 1, 16), jnp.float32),
    ],
)


def _sc_body(ei_hbm, tp_hbm, strg_hbm, m_hbm,
             accp_hbm,
             acc_sh, sblk, tblk, tpv0, tpv1, stv0, stv1, wov, mv,
             semA, semB, semS):
    c = lax.axis_index("c")
    s = lax.axis_index("s")
    w = s * NC + c
    tpvs, stvs, sems = (tpv0, tpv1), (stv0, stv1), (semA, semB)

    z16 = jnp.zeros((16,), jnp.float32)

    def zrow(r, _):
        for j in range(TW // 16):
            wov[r, pl.ds(j * 16, 16)] = z16
        return _

    lax.fori_loop(0, K, zrow, None)
    for b in range(-(-NZCH // NS)):
        cid = s + NS * b
        @pl.when(cid < NZCH)
        def _():
            ro = pl.multiple_of(cid * K, 8)
            pltpu.sync_copy(wov, acc_sh.at[pl.ds(ro, K)])
    pltpu.sync_copy(m_hbm, mv)
    plsc.subcore_barrier()

    iota16 = lax.iota(jnp.int32, 16)
    zero_i = iota16 * 0
    hidx = [(zero_i + h)[:, None] for h in range(H)]
    gdn = lax.GatherDimensionNumbers(
        offset_dims=(), collapsed_slice_dims=(0,), start_index_map=(0,))

    srow0 = w * NCH
    trow0 = SRCROWS + w * NCH

    def issue(lc, b):
        pltpu.async_copy(tp_hbm.at[sblk.at[lc]], tpvs[b], sems[b])
        pltpu.async_copy(strg_hbm.at[tblk.at[lc]], stvs[b], sems[b])

    def drain(lc, b):
        pltpu.make_async_copy(tp_hbm.at[sblk.at[lc]], tpvs[b], sems[b]).wait()
        pltpu.make_async_copy(strg_hbm.at[tblk.at[lc]], stvs[b], sems[b]).wait()

    def copy_idx(t):
        pltpu.sync_copy(ei_hbm.at[pl.ds(srow0 + t * SCW, SCW)], sblk)
        pltpu.sync_copy(ei_hbm.at[pl.ds(trow0 + t * SCW, SCW)], tblk)

    def compute(b):
        tpv, stv = tpvs[b], stvs[b]
        mreg = mv[...]

        def edge4(j4, _):
            for u in range(4):
                j = j4 * 4 + u
                z = tpv[j, pl.ds(HF, 16)] + stv[j, :]
                z = jnp.maximum(z, z * 0.2) - mreg
                e = jnp.exp(z)
                wov[j, pl.ds(HF, 16)] = e
                for h in range(H):
                    sp = lax.gather(
                        e, hidx[h], gdn, slice_sizes=(1,),
                        mode=lax.GatherScatterMode.PROMISE_IN_BOUNDS)
                    wov[j, pl.ds(h * 16, 16)] = tpv[j, pl.ds(h * 16, 16)] * sp
            return _

        lax.fori_loop(0, K // 4, edge4, None)

    copy_idx(0)
    issue(0, 0)

    def drain_scat(lc):
        pltpu.make_async_copy(wov, acc_sh.at[tblk.at[lc]], semS).wait()

    def superchunk(t, _):
        for lc in range(SCW):
            b = lc % 2
            drain(lc, b)
            if lc > 0:
                drain_scat(lc - 1)
            if lc + 1 < SCW:
                issue(lc + 1, 1 - b)
            compute(b)
            pltpu.async_copy(wov, acc_sh.at[tblk.at[lc]], semS, add=True)
            if lc + 1 == SCW:
                drain_scat(lc)
                @pl.when(t + 1 < NSC)
                def _():
                    copy_idx(t + 1)
                    issue(0, 1 - b)
        return _

    lax.fori_loop(0, NSC, superchunk, None)
    plsc.subcore_barrier()

    for b in range(2):
        cid = s + NS * b
        if (NS * b) < NCHK:
            @pl.when(cid < NCHK)
            def _():
                ro = pl.multiple_of(cid * CHK_R, 8)
                pltpu.sync_copy(acc_sh.at[pl.ds(ro, CHK_R)],
                                accp_hbm.at[c, pl.ds(ro, CHK_R)])


_sc_edge = functools.partial(
    pl.kernel,
    out_type=jax.ShapeDtypeStruct((NC, N, TW), jnp.float32),
    mesh=plsc.VectorSubcoreMesh(core_axis_name="c", subcore_axis_name="s"),
    compiler_params=pltpu.CompilerParams(use_tc_tiling_on_sc=False),
    scratch_types=[
        pltpu.VMEM_SHARED((NPAD, TW), jnp.float32),
        pltpu.VMEM((SCW, K), jnp.int32),
        pltpu.VMEM((SCW, K), jnp.int32),
        pltpu.VMEM((K, TW), jnp.float32),
        pltpu.VMEM((K, TW), jnp.float32),
        pltpu.VMEM((K, 16), jnp.float32),
        pltpu.VMEM((K, 16), jnp.float32),
        pltpu.VMEM((K, TW), jnp.float32),
        pltpu.VMEM((16,), jnp.float32),
        pltpu.SemaphoreType.DMA,
        pltpu.SemaphoreType.DMA,
        pltpu.SemaphoreType.DMA,
    ],
)(_sc_body)


def _epi_body(accp_ref, skip_ref, bias_ref, sel_ref, out_ref):
    a = accp_ref[0] + accp_ref[1]
    o = a[:, :HF]
    d = a[:, HF:]
    dexp = jnp.dot(d, sel_ref[...], preferred_element_type=jnp.float32) + 1e-16
    z = o / dexp + skip_ref[...] + bias_ref[...]
    out_ref[...] = jnp.where(z > 0, z, jnp.exp(jnp.minimum(z, 0.0)) - 1.0)


_epilogue = pl.pallas_call(
    _epi_body,
    grid=(GRID,),
    in_specs=[
        pl.BlockSpec((NC, BLK, TW), lambda i: (0, i, 0)),
        pl.BlockSpec((BLK, HF), lambda i: (i, 0)),
        pl.BlockSpec((1, HF), lambda i: (0, 0)),
        pl.BlockSpec((16, HF), lambda i: (0, 0)),
    ],
    out_specs=pl.BlockSpec((BLK, HF), lambda i: (i, 0)),
    out_shape=jax.ShapeDtypeStruct((N, HF), jnp.float32),
)


def kernel(x, edge_index, W_proj, a_src, a_trg, W_skip, bias):
    f32 = jnp.float32
    rows = jnp.arange(HF)
    cols = rows // F
    asrc_m = jnp.zeros((HF, 16), f32).at[rows, cols].set(a_src.reshape(HF))
    atrg_m = jnp.zeros((HF, 16), f32).at[rows, cols].set(a_trg.reshape(HF))
    sel16 = jnp.zeros((16, HF), f32).at[cols, rows].set(1.0)

    tp, skip, strg16, maxs, maxt = _prologue(
        x, W_proj.T, W_skip.T, asrc_m, atrg_m)

    msum = jnp.max(maxs) + jnp.max(maxt)
    m = jnp.maximum(msum, 0.2 * msum)
    m16 = jnp.full((16,), m, f32)

    npad = EPW_P - E // NW
    srcp = jnp.concatenate(
        [edge_index[0].reshape(NW, E // NW),
         jnp.zeros((NW, npad), jnp.int32)], axis=1)
    trgp = jnp.concatenate(
        [edge_index[1].reshape(NW, E // NW),
         jnp.full((NW, npad), N, jnp.int32)], axis=1)
    ei2 = jnp.concatenate([srcp.reshape(-1, K), trgp.reshape(-1, K)], axis=0)

    accp = _sc_edge(ei2, tp, strg16, m16)

    out = _epilogue(accp, skip, bias.reshape(1, HF), sel16)
    return (out, edge_index)


# speedup vs baseline: 1.1408x; 1.1403x over previous
import functools

import jax
import jax.numpy as jnp
from jax import lax
from jax.experimental import pallas as pl
from jax.experimental.pallas import tpu as pltpu
from jax.experimental.pallas import tpu_sc as plsc

N = 10000
E = 320000
D_IN = 128
H = 8
F = 16
HF = H * F
TW = HF + 16

NC = 2
NS = 16
NW = NC * NS
K = 80
SCW = 10
NCH = 130
NSC = NCH // SCW
EPW_P = NCH * K
SRCROWS = NW * NCH
NPAD = N + K
NZCH = NPAD // K
CHK_R = 400
NCHK = N // CHK_R

BLK = 1000
GRID = N // BLK


def _pro_body(x_ref, wp_ref, ws_ref, asrc_ref, atrg_ref,
              tp_ref, skip_ref, strg_ref, maxs_ref, maxt_ref):
    xb = x_ref[...]
    p = jnp.dot(xb, wp_ref[...], preferred_element_type=jnp.float32)
    skip_ref[...] = jnp.dot(xb, ws_ref[...], preferred_element_type=jnp.float32)
    ss = jnp.dot(p, asrc_ref[...], preferred_element_type=jnp.float32)
    st = jnp.dot(p, atrg_ref[...], preferred_element_type=jnp.float32)
    tp_ref[...] = jnp.concatenate([p, ss], axis=1)
    strg_ref[...] = st
    maxs_ref[...] = jnp.max(ss, axis=0).reshape(1, 1, 16)
    maxt_ref[...] = jnp.max(st, axis=0).reshape(1, 1, 16)


_prologue = pl.pallas_call(
    _pro_body,
    grid=(GRID,),
    in_specs=[
        pl.BlockSpec((BLK, D_IN), lambda i: (i, 0)),
        pl.BlockSpec((D_IN, HF), lambda i: (0, 0)),
        pl.BlockSpec((D_IN, HF), lambda i: (0, 0)),
        pl.BlockSpec((HF, 16), lambda i: (0, 0)),
        pl.BlockSpec((HF, 16), lambda i: (0, 0)),
    ],
    out_specs=[
        pl.BlockSpec((BLK, TW), lambda i: (i, 0)),
        pl.BlockSpec((BLK, HF), lambda i: (i, 0)),
        pl.BlockSpec((BLK, 16), lambda i: (i, 0)),
        pl.BlockSpec((1, 1, 16), lambda i: (i, 0, 0)),
        pl.BlockSpec((1, 1, 16), lambda i: (i, 0, 0)),
    ],
    out_shape=[
        jax.ShapeDtypeStruct((N, TW), jnp.float32),
        jax.ShapeDtypeStruct((N, HF), jnp.float32),
        jax.ShapeDtypeStruct((N, 16), jnp.float32),
        jax.ShapeDtypeStruct((GRID, 1, 16), jnp.float32),
        jax.ShapeDtypeStruct((GRID, 1, 16), jnp.float32),
    ],
)


def _sc_body(ei_hbm, tp_hbm, strg_hbm, m_hbm,
             accp_hbm,
             acc_sh, sblk, tblk, tpv0, tpv1, stv0, stv1, wov, mv,
             semA, semB, semS):
    c = lax.axis_index("c")
    s = lax.axis_index("s")
    w = s * NC + c
    tpvs, stvs, sems = (tpv0, tpv1), (stv0, stv1), (semA, semB)

    z16 = jnp.zeros((16,), jnp.float32)

    def zrow(r, _):
        for j in range(TW // 16):
            wov[r, pl.ds(j * 16, 16)] = z16
        return _

    lax.fori_loop(0, K, zrow, None)
    for b in range(-(-NZCH // NS)):
        cid = s + NS * b
        @pl.when(cid < NZCH)
        def _():
            ro = pl.multiple_of(cid * K, 8)
            pltpu.sync_copy(wov, acc_sh.at[pl.ds(ro, K)])
    pltpu.sync_copy(m_hbm, mv)
    plsc.subcore_barrier()

    iota16 = lax.iota(jnp.int32, 16)
    zero_i = iota16 * 0
    hidx = [(zero_i + h)[:, None] for h in range(H)]
    gdn = lax.GatherDimensionNumbers(
        offset_dims=(), collapsed_slice_dims=(0,), start_index_map=(0,))

    srow0 = w * NCH
    trow0 = SRCROWS + w * NCH

    def issue(lc, b):
        pltpu.async_copy(tp_hbm.at[sblk.at[lc]], tpvs[b], sems[b])
        pltpu.async_copy(strg_hbm.at[tblk.at[lc]], stvs[b], sems[b])

    def drain(lc, b):
        pltpu.make_async_copy(tp_hbm.at[sblk.at[lc]], tpvs[b], sems[b]).wait()
        pltpu.make_async_copy(strg_hbm.at[tblk.at[lc]], stvs[b], sems[b]).wait()

    def copy_idx(t):
        pltpu.sync_copy(ei_hbm.at[pl.ds(srow0 + t * SCW, SCW)], sblk)
        pltpu.sync_copy(ei_hbm.at[pl.ds(trow0 + t * SCW, SCW)], tblk)

    def compute(b):
        tpv, stv = tpvs[b], stvs[b]
        mreg = mv[...]

        def edge4(j4, _):
            es = []
            for u in range(4):
                j = j4 * 4 + u
                z = tpv[j, pl.ds(HF, 16)] + stv[j, :]
                z = jnp.maximum(z, z * 0.2) - mreg
                es.append(jnp.exp(z))
            for u in range(4):
                j = j4 * 4 + u
                e = es[u]
                wov[j, pl.ds(HF, 16)] = e
                for h in range(H):
                    sp = lax.gather(
                        e, hidx[h], gdn, slice_sizes=(1,),
                        mode=lax.GatherScatterMode.PROMISE_IN_BOUNDS)
                    wov[j, pl.ds(h * 16, 16)] = tpv[j, pl.ds(h * 16, 16)] * sp
            return _

        lax.fori_loop(0, K // 4, edge4, None)

    copy_idx(0)
    issue(0, 0)

    def drain_scat(lc):
        pltpu.make_async_copy(wov, acc_sh.at[tblk.at[lc]], semS).wait()

    def superchunk(t, _):
        for lc in range(SCW):
            b = lc % 2
            drain(lc, b)
            if lc > 0:
                drain_scat(lc - 1)
            if lc + 1 < SCW:
                issue(lc + 1, 1 - b)
            compute(b)
            pltpu.async_copy(wov, acc_sh.at[tblk.at[lc]], semS, add=True)
            if lc + 1 == SCW:
                drain_scat(lc)
                @pl.when(t + 1 < NSC)
                def _():
                    copy_idx(t + 1)
                    issue(0, 1 - b)
        return _

    lax.fori_loop(0, NSC, superchunk, None)
    plsc.subcore_barrier()

    for b in range(2):
        cid = s + NS * b
        if (NS * b) < NCHK:
            @pl.when(cid < NCHK)
            def _():
                ro = pl.multiple_of(cid * CHK_R, 8)
                pltpu.sync_copy(acc_sh.at[pl.ds(ro, CHK_R)],
                                accp_hbm.at[c, pl.ds(ro, CHK_R)])


_sc_edge = functools.partial(
    pl.kernel,
    out_type=jax.ShapeDtypeStruct((NC, N, TW), jnp.float32),
    mesh=plsc.VectorSubcoreMesh(core_axis_name="c", subcore_axis_name="s"),
    compiler_params=pltpu.CompilerParams(use_tc_tiling_on_sc=False),
    scratch_types=[
        pltpu.VMEM_SHARED((NPAD, TW), jnp.float32),
        pltpu.VMEM((SCW, K), jnp.int32),
        pltpu.VMEM((SCW, K), jnp.int32),
        pltpu.VMEM((K, TW), jnp.float32),
        pltpu.VMEM((K, TW), jnp.float32),
        pltpu.VMEM((K, 16), jnp.float32),
        pltpu.VMEM((K, 16), jnp.float32),
        pltpu.VMEM((K, TW), jnp.float32),
        pltpu.VMEM((16,), jnp.float32),
        pltpu.SemaphoreType.DMA,
        pltpu.SemaphoreType.DMA,
        pltpu.SemaphoreType.DMA,
    ],
)(_sc_body)


def _epi_body(accp_ref, skip_ref, bias_ref, sel_ref, out_ref):
    a = accp_ref[0] + accp_ref[1]
    o = a[:, :HF]
    d = a[:, HF:]
    dexp = jnp.dot(d, sel_ref[...], preferred_element_type=jnp.float32) + 1e-16
    z = o / dexp + skip_ref[...] + bias_ref[...]
    out_ref[...] = jnp.where(z > 0, z, jnp.exp(jnp.minimum(z, 0.0)) - 1.0)


_epilogue = pl.pallas_call(
    _epi_body,
    grid=(GRID,),
    in_specs=[
        pl.BlockSpec((NC, BLK, TW), lambda i: (0, i, 0)),
        pl.BlockSpec((BLK, HF), lambda i: (i, 0)),
        pl.BlockSpec((1, HF), lambda i: (0, 0)),
        pl.BlockSpec((16, HF), lambda i: (0, 0)),
    ],
    out_specs=pl.BlockSpec((BLK, HF), lambda i: (i, 0)),
    out_shape=jax.ShapeDtypeStruct((N, HF), jnp.float32),
)


def kernel(x, edge_index, W_proj, a_src, a_trg, W_skip, bias):
    f32 = jnp.float32
    rows = jnp.arange(HF)
    cols = rows // F
    asrc_m = jnp.zeros((HF, 16), f32).at[rows, cols].set(a_src.reshape(HF))
    atrg_m = jnp.zeros((HF, 16), f32).at[rows, cols].set(a_trg.reshape(HF))
    sel16 = jnp.zeros((16, HF), f32).at[cols, rows].set(1.0)

    tp, skip, strg16, maxs, maxt = _prologue(
        x, W_proj.T, W_skip.T, asrc_m, atrg_m)

    msum = jnp.max(maxs) + jnp.max(maxt)
    m = jnp.maximum(msum, 0.2 * msum)
    m16 = jnp.full((16,), m, f32)

    npad = EPW_P - E // NW
    srcp = jnp.concatenate(
        [edge_index[0].reshape(NW, E // NW),
         jnp.zeros((NW, npad), jnp.int32)], axis=1)
    trgp = jnp.concatenate(
        [edge_index[1].reshape(NW, E // NW),
         jnp.full((NW, npad), N, jnp.int32)], axis=1)
    ei2 = jnp.concatenate([srcp.reshape(-1, K), trgp.reshape(-1, K)], axis=0)

    accp = _sc_edge(ei2, tp, strg16, m16)

    out = _epilogue(accp, skip, bias.reshape(1, HF), sel16)
    return (out, edge_index)


# speedup vs baseline: 1.1522x; 1.0100x over previous
import functools

import jax
import jax.numpy as jnp
from jax import lax
from jax.experimental import pallas as pl
from jax.experimental.pallas import tpu as pltpu
from jax.experimental.pallas import tpu_sc as plsc

N = 10000
E = 320000
D_IN = 128
H = 8
F = 16
HF = H * F
TW = HF + 16

NC = 2
NS = 16
NW = NC * NS
K = 80
SCW = 10
NCH = 130
NSC = NCH // SCW
EPW_P = NCH * K
SRCROWS = NW * NCH
NPAD = N + K
NZCH = NPAD // K
CHK_R = 400
NCHK = N // CHK_R

BLK = 1000
GRID = N // BLK


def _pro_body(x_ref, wp_ref, ws_ref, asrc_ref, atrg_ref,
              tp_ref, skip_ref, strg_ref, maxs_ref, maxt_ref):
    xb = x_ref[...]
    p = jnp.dot(xb, wp_ref[...], preferred_element_type=jnp.float32)
    skip_ref[...] = jnp.dot(xb, ws_ref[...], preferred_element_type=jnp.float32)
    ss = jnp.dot(p, asrc_ref[...], preferred_element_type=jnp.float32)
    st = jnp.dot(p, atrg_ref[...], preferred_element_type=jnp.float32)
    tp_ref[...] = jnp.concatenate([p, ss], axis=1)
    strg_ref[...] = st
    maxs_ref[...] = jnp.max(ss, axis=0).reshape(1, 1, 16)
    maxt_ref[...] = jnp.max(st, axis=0).reshape(1, 1, 16)


_prologue = pl.pallas_call(
    _pro_body,
    grid=(GRID,),
    in_specs=[
        pl.BlockSpec((BLK, D_IN), lambda i: (i, 0)),
        pl.BlockSpec((D_IN, HF), lambda i: (0, 0)),
        pl.BlockSpec((D_IN, HF), lambda i: (0, 0)),
        pl.BlockSpec((HF, 16), lambda i: (0, 0)),
        pl.BlockSpec((HF, 16), lambda i: (0, 0)),
    ],
    out_specs=[
        pl.BlockSpec((BLK, TW), lambda i: (i, 0)),
        pl.BlockSpec((BLK, HF), lambda i: (i, 0)),
        pl.BlockSpec((BLK, 16), lambda i: (i, 0)),
        pl.BlockSpec((1, 1, 16), lambda i: (i, 0, 0)),
        pl.BlockSpec((1, 1, 16), lambda i: (i, 0, 0)),
    ],
    out_shape=[
        jax.ShapeDtypeStruct((N, TW), jnp.float32),
        jax.ShapeDtypeStruct((N, HF), jnp.float32),
        jax.ShapeDtypeStruct((N, 16), jnp.float32),
        jax.ShapeDtypeStruct((GRID, 1, 16), jnp.float32),
        jax.ShapeDtypeStruct((GRID, 1, 16), jnp.float32),
    ],
)


def _sc_body(ei_hbm, tp_hbm, strg_hbm, m_hbm,
             accp_hbm,
             acc_sh, sblk, tblk, tpv0, tpv1, stv0, stv1, wov, mv,
             semA, semB, semS):
    c = lax.axis_index("c")
    s = lax.axis_index("s")
    w = s * NC + c
    tpvs, stvs, sems = (tpv0, tpv1), (stv0, stv1), (semA, semB)

    z16 = jnp.zeros((16,), jnp.float32)

    def zrow(r, _):
        for j in range(TW // 16):
            wov[r, pl.ds(j * 16, 16)] = z16
        return _

    lax.fori_loop(0, K, zrow, None)
    for b in range(-(-NZCH // NS)):
        cid = s + NS * b
        @pl.when(cid < NZCH)
        def _():
            ro = pl.multiple_of(cid * K, 8)
            pltpu.sync_copy(wov, acc_sh.at[pl.ds(ro, K)])
    pltpu.sync_copy(m_hbm, mv)
    plsc.subcore_barrier()

    iota16 = lax.iota(jnp.int32, 16)
    zero_i = iota16 * 0
    hidx = [(zero_i + h)[:, None] for h in range(H)]
    gdn = lax.GatherDimensionNumbers(
        offset_dims=(), collapsed_slice_dims=(0,), start_index_map=(0,))

    srow0 = w * NCH
    trow0 = SRCROWS + w * NCH

    def issue(lc, b):
        pltpu.async_copy(tp_hbm.at[sblk.at[lc]], tpvs[b], sems[b])
        pltpu.async_copy(strg_hbm.at[tblk.at[lc]], stvs[b], sems[b])

    def drain(lc, b):
        pltpu.make_async_copy(tp_hbm.at[sblk.at[lc]], tpvs[b], sems[b]).wait()
        pltpu.make_async_copy(strg_hbm.at[tblk.at[lc]], stvs[b], sems[b]).wait()

    def copy_idx(t):
        pltpu.sync_copy(ei_hbm.at[pl.ds(srow0 + t * SCW, SCW)], sblk)
        pltpu.sync_copy(ei_hbm.at[pl.ds(trow0 + t * SCW, SCW)], tblk)

    def compute(b):
        tpv, stv = tpvs[b], stvs[b]
        mreg = mv[...]

        def edge4(j4, _):
            es = []
            for u in range(8):
                j = j4 * 8 + u
                z = tpv[j, pl.ds(HF, 16)] + stv[j, :]
                z = jnp.maximum(z, z * 0.2) - mreg
                es.append(jnp.exp(z))
            for u in range(8):
                j = j4 * 8 + u
                e = es[u]
                wov[j, pl.ds(HF, 16)] = e
                for h in range(H):
                    sp = lax.gather(
                        e, hidx[h], gdn, slice_sizes=(1,),
                        mode=lax.GatherScatterMode.PROMISE_IN_BOUNDS)
                    wov[j, pl.ds(h * 16, 16)] = tpv[j, pl.ds(h * 16, 16)] * sp
            return _

        lax.fori_loop(0, K // 8, edge4, None)

    copy_idx(0)
    issue(0, 0)

    def drain_scat(lc):
        pltpu.make_async_copy(wov, acc_sh.at[tblk.at[lc]], semS).wait()

    def superchunk(t, _):
        for lc in range(SCW):
            b = lc % 2
            drain(lc, b)
            if lc > 0:
                drain_scat(lc - 1)
            if lc + 1 < SCW:
                issue(lc + 1, 1 - b)
            compute(b)
            pltpu.async_copy(wov, acc_sh.at[tblk.at[lc]], semS, add=True)
            if lc + 1 == SCW:
                drain_scat(lc)
                @pl.when(t + 1 < NSC)
                def _():
                    copy_idx(t + 1)
                    issue(0, 1 - b)
        return _

    lax.fori_loop(0, NSC, superchunk, None)
    plsc.subcore_barrier()

    for b in range(2):
        cid = s + NS * b
        if (NS * b) < NCHK:
            @pl.when(cid < NCHK)
            def _():
                ro = pl.multiple_of(cid * CHK_R, 8)
                pltpu.sync_copy(acc_sh.at[pl.ds(ro, CHK_R)],
                                accp_hbm.at[c, pl.ds(ro, CHK_R)])


_sc_edge = functools.partial(
    pl.kernel,
    out_type=jax.ShapeDtypeStruct((NC, N, TW), jnp.float32),
    mesh=plsc.VectorSubcoreMesh(core_axis_name="c", subcore_axis_name="s"),
    compiler_params=pltpu.CompilerParams(use_tc_tiling_on_sc=False),
    scratch_types=[
        pltpu.VMEM_SHARED((NPAD, TW), jnp.float32),
        pltpu.VMEM((SCW, K), jnp.int32),
        pltpu.VMEM((SCW, K), jnp.int32),
        pltpu.VMEM((K, TW), jnp.float32),
        pltpu.VMEM((K, TW), jnp.float32),
        pltpu.VMEM((K, 16), jnp.float32),
        pltpu.VMEM((K, 16), jnp.float32),
        pltpu.VMEM((K, TW), jnp.float32),
        pltpu.VMEM((16,), jnp.float32),
        pltpu.SemaphoreType.DMA,
        pltpu.SemaphoreType.DMA,
        pltpu.SemaphoreType.DMA,
    ],
)(_sc_body)


def _epi_body(accp_ref, skip_ref, bias_ref, sel_ref, out_ref):
    a = accp_ref[0] + accp_ref[1]
    o = a[:, :HF]
    d = a[:, HF:]
    dexp = jnp.dot(d, sel_ref[...], preferred_element_type=jnp.float32) + 1e-16
    z = o / dexp + skip_ref[...] + bias_ref[...]
    out_ref[...] = jnp.where(z > 0, z, jnp.exp(jnp.minimum(z, 0.0)) - 1.0)


_epilogue = pl.pallas_call(
    _epi_body,
    grid=(GRID,),
    in_specs=[
        pl.BlockSpec((NC, BLK, TW), lambda i: (0, i, 0)),
        pl.BlockSpec((BLK, HF), lambda i: (i, 0)),
        pl.BlockSpec((1, HF), lambda i: (0, 0)),
        pl.BlockSpec((16, HF), lambda i: (0, 0)),
    ],
    out_specs=pl.BlockSpec((BLK, HF), lambda i: (i, 0)),
    out_shape=jax.ShapeDtypeStruct((N, HF), jnp.float32),
)


def kernel(x, edge_index, W_proj, a_src, a_trg, W_skip, bias):
    f32 = jnp.float32
    rows = jnp.arange(HF)
    cols = rows // F
    asrc_m = jnp.zeros((HF, 16), f32).at[rows, cols].set(a_src.reshape(HF))
    atrg_m = jnp.zeros((HF, 16), f32).at[rows, cols].set(a_trg.reshape(HF))
    sel16 = jnp.zeros((16, HF), f32).at[cols, rows].set(1.0)

    tp, skip, strg16, maxs, maxt = _prologue(
        x, W_proj.T, W_skip.T, asrc_m, atrg_m)

    msum = jnp.max(maxs) + jnp.max(maxt)
    m = jnp.maximum(msum, 0.2 * msum)
    m16 = jnp.full((16,), m, f32)

    npad = EPW_P - E // NW
    srcp = jnp.concatenate(
        [edge_index[0].reshape(NW, E // NW),
         jnp.zeros((NW, npad), jnp.int32)], axis=1)
    trgp = jnp.concatenate(
        [edge_index[1].reshape(NW, E // NW),
         jnp.full((NW, npad), N, jnp.int32)], axis=1)
    ei2 = jnp.concatenate([srcp.reshape(-1, K), trgp.reshape(-1, K)], axis=0)

    accp = _sc_edge(ei2, tp, strg16, m16)

    out = _epilogue(accp, skip, bias.reshape(1, HF), sel16)
    return (out, edge_index)
